# baseline (speedup 1.0000x reference)
import functools

import jax
import jax.numpy as jnp
from jax import lax
from jax.experimental import pallas as pl
from jax.experimental.pallas import tpu as pltpu
from jax.experimental.pallas import tpu_sc as plsc

EMB = 64
NUM_PRED = 4
N_ATOMS_PER_PRED = 4096
TOTAL = NUM_PRED * N_ATOMS_PER_PRED
NC, NS, L = 2, 16, 16
NW = NC * NS
APW = TOTAL // NW
WPP = N_ATOMS_PER_PRED // APW
CHUNK = 128
NCHUNK = 2 * APW // CHUNK

_MESH = plsc.VectorSubcoreMesh(core_axis_name="c", subcore_axis_name="s")


@functools.partial(
    pl.kernel,
    out_type=(
        jax.ShapeDtypeStruct((NW, APW, EMB), jnp.float32),
        jax.ShapeDtypeStruct((NW, APW), jnp.float32),
    ),
    mesh=_MESH,
    compiler_params=pltpu.CompilerParams(use_tc_tiling_on_sc=False),
    scratch_types=[
        pltpu.VMEM((NCHUNK, CHUNK), jnp.int32),
        pltpu.VMEM((NCHUNK, CHUNK), jnp.int32),
        pltpu.VMEM((2 * APW, EMB), jnp.float32),
        pltpu.VMEM((APW, EMB), jnp.float32),
        pltpu.VMEM((APW,), jnp.float32),
        pltpu.VMEM((EMB,), jnp.float32),
        pltpu.SemaphoreType.DMA,
    ],
)
def _sc_kernel(x_hbm, a0_hbm, a1_hbm, a2_hbm, a3_hbm, ctab_hbm, ptab_hbm,
               emb_hbm, scores_hbm,
               a_v, xv_v, rows_v, emb_v, scores_v, pred_v, sem):
    wid = lax.axis_index("s") * NC + lax.axis_index("c")
    p = wid // WPP
    sub = wid % WPP

    for i, ap in enumerate((a0_hbm, a1_hbm, a2_hbm, a3_hbm)):
        @pl.when(p == i)
        def _():
            pltpu.sync_copy(ap.at[pl.ds(sub * NCHUNK, NCHUNK)], a_v)
    pltpu.sync_copy(ptab_hbm.at[p], pred_v)

    cps = [pltpu.async_copy(x_hbm.at[a_v.at[j]], xv_v.at[j], sem)
           for j in range(NCHUNK)]
    for c in cps:
        c.wait()

    cps = [pltpu.async_copy(ctab_hbm.at[xv_v.at[j]],
                            rows_v.at[pl.ds(j * CHUNK, CHUNK)], sem)
           for j in range(NCHUNK)]
    for c in cps:
        c.wait()

    pk = [pred_v[pl.ds(k * L, L)] for k in range(EMB // L)]
    lane = lax.iota(jnp.int32, L)

    @pl.loop(0, APW // L)
    def _group(g):
        score_vec = jnp.zeros((L,), jnp.float32)
        for a16 in range(L):
            a = g * L + a16
            s = None
            for k in range(EMB // L):
                e = (pk[k] * rows_v[2 * a, pl.ds(k * L, L)]
                     * rows_v[2 * a + 1, pl.ds(k * L, L)])
                emb_v[a, pl.ds(k * L, L)] = e
                s = e if s is None else s + e
            for b in range(4):
                s = s + s.at[lane ^ (1 << b)].get(mode="promise_in_bounds")
            score_vec = jnp.where(lane == a16, s, score_vec)
        scores_v[pl.ds(g * L, L)] = 1.0 / (1.0 + jnp.exp(-score_vec))

    pltpu.sync_copy(emb_v, emb_hbm.at[wid])
    pltpu.sync_copy(scores_v, scores_hbm.at[wid])


def kernel(X_domain_entity, A_pred0, A_pred1, A_pred2, A_pred3,
           constant_table, predicate_table):
    aps = [a.astype(jnp.int32).reshape(WPP * NCHUNK, CHUNK)
           for a in (A_pred0, A_pred1, A_pred2, A_pred3)]
    x = X_domain_entity.astype(jnp.int32)
    emb, scores = _sc_kernel(x, *aps, constant_table, predicate_table)
    atom_embeddings = emb.reshape(TOTAL, EMB)
    atom_outputs = scores.reshape(TOTAL, 1, 1)
    return (atom_outputs, atom_embeddings)


# speedup vs baseline: 1.0554x; 1.0554x over previous
import functools

import jax
import jax.numpy as jnp
from jax import lax
from jax.experimental import pallas as pl
from jax.experimental.pallas import tpu as pltpu
from jax.experimental.pallas import tpu_sc as plsc

EMB = 64
NUM_PRED = 4
N_ATOMS_PER_PRED = 4096
TOTAL = NUM_PRED * N_ATOMS_PER_PRED
NC, NS, L = 2, 16, 16
NW = NC * NS
APW = TOTAL // NW
CHUNK = 128
NCHUNK = APW // CHUNK
GPC = CHUNK // L

_MESH = plsc.VectorSubcoreMesh(core_axis_name="c", subcore_axis_name="s")


@functools.partial(
    pl.kernel,
    out_type=(
        jax.ShapeDtypeStruct((NW * NCHUNK, CHUNK, EMB), jnp.float32),
        jax.ShapeDtypeStruct((NW, APW), jnp.float32),
    ),
    mesh=_MESH,
    compiler_params=pltpu.CompilerParams(use_tc_tiling_on_sc=False),
    scratch_types=[
        pltpu.VMEM((NCHUNK, CHUNK), jnp.int32),
        pltpu.VMEM((NCHUNK, CHUNK), jnp.int32),
        pltpu.VMEM((NCHUNK, CHUNK), jnp.int32),
        pltpu.VMEM((NCHUNK, CHUNK), jnp.int32),
        pltpu.VMEM((APW, EMB), jnp.float32),
        pltpu.VMEM((APW, EMB), jnp.float32),
        pltpu.VMEM((APW, EMB), jnp.float32),
        pltpu.VMEM((APW,), jnp.float32),
        pltpu.VMEM((EMB,), jnp.float32),
        pltpu.SemaphoreType.DMA((2 * NCHUNK,)),
        pltpu.SemaphoreType.DMA((2 * NCHUNK,)),
        pltpu.SemaphoreType.DMA((NCHUNK,)),
    ],
)
def _sc_kernel(x_hbm, idxh_hbm, idxt_hbm, ctab_hbm, ptab_hbm,
               emb_hbm, scores_hbm,
               idxh_v, idxt_v, xh_v, xt_v, rows_h, rows_t, emb_v,
               scores_v, pred_v, sem1, sem2, sem3):
    wid = lax.axis_index("s") * NC + lax.axis_index("c")
    base = wid * APW

    pltpu.sync_copy(idxh_hbm.at[wid], idxh_v)
    pltpu.sync_copy(idxt_hbm.at[wid], idxt_v)
    p = wid // (N_ATOMS_PER_PRED // APW)
    pltpu.sync_copy(ptab_hbm.at[p], pred_v)

    st1 = []
    for j in range(NCHUNK):
        st1.append(pltpu.async_copy(x_hbm.at[idxh_v.at[j]], xh_v.at[j],
                                    sem1.at[2 * j]))
        st1.append(pltpu.async_copy(x_hbm.at[idxt_v.at[j]], xt_v.at[j],
                                    sem1.at[2 * j + 1]))
    st2 = []
    for j in range(NCHUNK):
        st1[2 * j].wait()
        st2.append(pltpu.async_copy(
            ctab_hbm.at[xh_v.at[j]], rows_h.at[pl.ds(j * CHUNK, CHUNK)],
            sem2.at[2 * j]))
        st1[2 * j + 1].wait()
        st2.append(pltpu.async_copy(
            ctab_hbm.at[xt_v.at[j]], rows_t.at[pl.ds(j * CHUNK, CHUNK)],
            sem2.at[2 * j + 1]))

    pk = [pred_v[pl.ds(k * L, L)] for k in range(EMB // L)]
    lane = lax.iota(jnp.int32, L)
    st3 = []
    for j in range(NCHUNK):
        st2[2 * j].wait()
        st2[2 * j + 1].wait()

        @pl.loop(j * GPC, (j + 1) * GPC)
        def _group(g):
            score_vec = jnp.zeros((L,), jnp.float32)
            for a16 in range(L):
                a = g * L + a16
                s = None
                for k in range(EMB // L):
                    e = (pk[k] * rows_h[a, pl.ds(k * L, L)]
                         * rows_t[a, pl.ds(k * L, L)])
                    emb_v[a, pl.ds(k * L, L)] = e
                    s = e if s is None else s + e
                for b in range(4):
                    s = s + s.at[lane ^ (1 << b)].get(mode="promise_in_bounds")
                score_vec = jnp.where(lane == a16, s, score_vec)
            scores_v[pl.ds(g * L, L)] = 1.0 / (1.0 + jnp.exp(-score_vec))

        st3.append(pltpu.async_copy(
            emb_v.at[pl.ds(j * CHUNK, CHUNK)],
            emb_hbm.at[wid * NCHUNK + j], sem3.at[j]))

    pltpu.sync_copy(scores_v, scores_hbm.at[wid])
    for c in st3:
        c.wait()


def kernel(X_domain_entity, A_pred0, A_pred1, A_pred2, A_pred3,
           constant_table, predicate_table):
    A = jnp.concatenate([A_pred0, A_pred1, A_pred2, A_pred3], axis=0)
    idx_h = A[:, 0].astype(jnp.int32).reshape(NW, NCHUNK, CHUNK)
    idx_t = A[:, 1].astype(jnp.int32).reshape(NW, NCHUNK, CHUNK)
    x = X_domain_entity.astype(jnp.int32)
    emb, scores = _sc_kernel(x, idx_h, idx_t, constant_table,
                             predicate_table)
    atom_embeddings = emb.reshape(TOTAL, EMB)
    atom_outputs = scores.reshape(TOTAL, 1, 1)
    return (atom_outputs, atom_embeddings)


# speedup vs baseline: 1.0707x; 1.0145x over previous
import functools

import jax
import jax.numpy as jnp
from jax import lax
from jax.experimental import pallas as pl
from jax.experimental.pallas import tpu as pltpu
from jax.experimental.pallas import tpu_sc as plsc

EMB = 64
NUM_PRED = 4
N_ATOMS_PER_PRED = 4096
TOTAL = NUM_PRED * N_ATOMS_PER_PRED
NC, NS, L = 2, 16, 16
NW = NC * NS
APW = TOTAL // NW
CHUNK = 128
NCHUNK = APW // CHUNK

_MESH = plsc.VectorSubcoreMesh(core_axis_name="c", subcore_axis_name="s")


@functools.partial(
    pl.kernel,
    out_type=(
        jax.ShapeDtypeStruct((NW, APW, EMB), jnp.float32),
        jax.ShapeDtypeStruct((NW, APW), jnp.float32),
    ),
    mesh=_MESH,
    compiler_params=pltpu.CompilerParams(use_tc_tiling_on_sc=False),
    scratch_types=[
        pltpu.VMEM((NCHUNK, CHUNK), jnp.int32),
        pltpu.VMEM((NCHUNK, CHUNK), jnp.int32),
        pltpu.VMEM((NCHUNK, CHUNK), jnp.int32),
        pltpu.VMEM((NCHUNK, CHUNK), jnp.int32),
        pltpu.VMEM((APW, EMB), jnp.float32),
        pltpu.VMEM((APW, EMB), jnp.float32),
        pltpu.VMEM((APW, EMB), jnp.float32),
        pltpu.VMEM((APW,), jnp.float32),
        pltpu.VMEM((EMB,), jnp.float32),
        pltpu.SemaphoreType.DMA,
    ],
)
def _sc_kernel(x_hbm, idxh_hbm, idxt_hbm, ctab_hbm, ptab_hbm,
               emb_hbm, scores_hbm,
               idxh_v, idxt_v, xh_v, xt_v, rows_h, rows_t, emb_v,
               scores_v, pred_v, sem):
    wid = lax.axis_index("s") * NC + lax.axis_index("c")

    pltpu.sync_copy(idxh_hbm.at[wid], idxh_v)
    pltpu.sync_copy(idxt_hbm.at[wid], idxt_v)
    p = wid // (N_ATOMS_PER_PRED // APW)
    pltpu.sync_copy(ptab_hbm.at[p], pred_v)

    cps = []
    for j in range(NCHUNK):
        cps.append(pltpu.async_copy(x_hbm.at[idxh_v.at[j]], xh_v.at[j], sem))
        cps.append(pltpu.async_copy(x_hbm.at[idxt_v.at[j]], xt_v.at[j], sem))
    for c in cps:
        c.wait()

    cps = []
    for j in range(NCHUNK):
        cps.append(pltpu.async_copy(
            ctab_hbm.at[xh_v.at[j]], rows_h.at[pl.ds(j * CHUNK, CHUNK)], sem))
        cps.append(pltpu.async_copy(
            ctab_hbm.at[xt_v.at[j]], rows_t.at[pl.ds(j * CHUNK, CHUNK)], sem))
    for c in cps:
        c.wait()

    pk = [pred_v[pl.ds(k * L, L)] for k in range(EMB // L)]
    lane = lax.iota(jnp.int32, L)

    @pl.loop(0, APW // L)
    def _group(g):
        score_vec = jnp.zeros((L,), jnp.float32)
        for a16 in range(L):
            a = g * L + a16
            s = None
            for k in range(EMB // L):
                e = pk[k] * rows_h[a, pl.ds(k * L, L)] * rows_t[a, pl.ds(k * L, L)]
                emb_v[a, pl.ds(k * L, L)] = e
                s = e if s is None else s + e
            for b in range(4):
                s = s + s.at[lane ^ (1 << b)].get(mode="promise_in_bounds")
            score_vec = jnp.where(lane == a16, s, score_vec)
        scores_v[pl.ds(g * L, L)] = 1.0 / (1.0 + jnp.exp(-score_vec))

    pltpu.sync_copy(emb_v, emb_hbm.at[wid])
    pltpu.sync_copy(scores_v, scores_hbm.at[wid])


def kernel(X_domain_entity, A_pred0, A_pred1, A_pred2, A_pred3,
           constant_table, predicate_table):
    A = jnp.concatenate([A_pred0, A_pred1, A_pred2, A_pred3], axis=0)
    idx_h = A[:, 0].astype(jnp.int32).reshape(NW, NCHUNK, CHUNK)
    idx_t = A[:, 1].astype(jnp.int32).reshape(NW, NCHUNK, CHUNK)
    x = X_domain_entity.astype(jnp.int32)
    emb, scores = _sc_kernel(x, idx_h, idx_t, constant_table, predicate_table)
    atom_embeddings = emb.reshape(TOTAL, EMB)
    atom_outputs = scores.reshape(TOTAL, 1, 1)
    return (atom_outputs, atom_embeddings)
